# baseline (speedup 1.0000x reference)
import functools

import jax
import jax.numpy as jnp
from jax import lax
from jax.experimental import pallas as pl
from jax.experimental.pallas import tpu as pltpu
from jax.experimental.pallas import tpu_sc as plsc

_FIGSIZE = 416.0
_CONF_THRESH = 0.2
_B = 16
_N = 20000
_HALF = _N // 2
_CB = 2000
_NCH = _HALF // _CB
_GROUPS = _CB // 16

_LN2 = 0.6931471805599453
_SQRT2 = 1.4142135623730951


def _splat(val, dtype=jnp.float32):
    return jnp.full((16,), val, dtype)


def _log_poly(t):
    bits = lax.bitcast_convert_type(t, jnp.int32)
    e = (bits >> 23) - 127
    m = lax.bitcast_convert_type(
        (bits & 0x007FFFFF) | 0x3F800000, jnp.float32
    )
    big = m > _SQRT2
    m = jnp.where(big, m * 0.5, m)
    ef = e.astype(jnp.float32) + jnp.where(big, 1.0, 0.0)
    z = (m - 1.0) / (m + 1.0)
    z2 = z * z
    q = (0.14285714 * z2 + 0.2) * z2 + 0.33333333
    log_m = (2.0 * z) * (1.0 + z2 * q)
    return ef * _LN2 + log_m


def _worker_body(boxes_hbm, gt_hbm, out_hbm, buf0, buf1, gtv, stg, sem0, sem1):
    img = lax.axis_index("s")
    half = lax.axis_index("c")
    wid = img * 2 + half
    woff = pl.multiple_of(img * (_N * 7) + half * (_HALF * 7), 8)

    pltpu.sync_copy(gt_hbm.at[img], gtv)
    gx1 = gtv[0]
    gy1 = gtv[1]
    gx2 = gtv[2]
    gy2 = gtv[3]
    area_g = gtv[4]
    thr = gtv[5]

    iota7 = lax.iota(jnp.int32, 16) * 7

    bufs = (buf0, buf1)
    sems = (sem0, sem1)
    copies = [None, None]

    def start(c):
        copies[c % 2] = pltpu.async_copy(
            boxes_hbm.at[pl.ds(pl.multiple_of(woff + c * (_CB * 7), 8), _CB * 7)],
            bufs[c % 2],
            sems[c % 2],
        )

    def chunk_body(buf, accs):
        def group(g, accs):
            det, cnt, scf = accs
            base = iota7 + g * 112
            x = plsc.load_gather(buf, [base])
            y = plsc.load_gather(buf, [base + 1])
            w = plsc.load_gather(buf, [base + 2])
            h = plsc.load_gather(buf, [base + 3])
            cf = plsc.load_gather(buf, [base + 4])
            cl = plsc.load_gather(buf, [base + 6])

            wh = w * 0.5
            hh = h * 0.5
            bx1 = x - wh
            by1 = y - hh
            bx2 = x + wh
            by2 = y + hh
            iw = jnp.maximum(jnp.minimum(bx2, gx2) - jnp.maximum(bx1, gx1), 0.0)
            ih = jnp.maximum(jnp.minimum(by2, gy2) - jnp.maximum(by1, gy1), 0.0)
            inter = iw * ih
            union = w * h + area_g - inter
            iou = inter / union
            valid = (iou >= thr) & (cl == 0.0) & (cf > _CONF_THRESH)
            s = jnp.minimum(cf, 1.0 - 1e-6)
            lg = _log_poly(1.0 - s)
            zero = jnp.zeros((16,), jnp.float32)
            det = det - jnp.where(valid, lg * iou, zero)
            cnt = cnt + jnp.where(valid, 1.0, 0.0)
            scf = scf + jnp.where(valid, cf, zero)
            return det, cnt, scf

        return lax.fori_loop(0, _GROUPS, group, accs)

    accs = (
        jnp.zeros((16,), jnp.float32),
        jnp.zeros((16,), jnp.float32),
        jnp.zeros((16,), jnp.float32),
    )
    start(0)
    for c in range(_NCH):
        if c + 1 < _NCH:
            start(c + 1)
        copies[c % 2].wait()
        accs = chunk_body(bufs[c % 2], accs)

    stg[0, :] = accs[0]
    stg[1, :] = accs[1]
    stg[2, :] = accs[2]
    pltpu.sync_copy(stg, out_hbm.at[wid])


def kernel(boxes, gt, iou_thresh):
    gtn = gt * (1.0 / _FIGSIZE)
    area_g = (gtn[:, 2] - gtn[:, 0]) * (gtn[:, 3] - gtn[:, 1])
    thr = jnp.broadcast_to(jnp.asarray(iou_thresh, jnp.float32), (_B,))
    gt_rows = jnp.concatenate(
        [gtn, area_g[:, None], thr[:, None]], axis=1
    )
    gt_exp = jnp.broadcast_to(gt_rows[:, :, None], (_B, 6, 16)).astype(jnp.float32)

    mesh = plsc.VectorSubcoreMesh(core_axis_name="c", subcore_axis_name="s")
    boxes_flat = jnp.reshape(boxes, (_B * _N * 7,))
    sc_call = functools.partial(
        pl.kernel,
        mesh=mesh,
        out_type=jax.ShapeDtypeStruct((32, 3, 16), jnp.float32),
        scratch_types=[
            pltpu.VMEM((_CB * 7,), jnp.float32),
            pltpu.VMEM((_CB * 7,), jnp.float32),
            pltpu.VMEM((6, 16), jnp.float32),
            pltpu.VMEM((3, 16), jnp.float32),
            pltpu.SemaphoreType.DMA,
            pltpu.SemaphoreType.DMA,
        ],
        compiler_params=pltpu.CompilerParams(needs_layout_passes=False),
    )(_worker_body)
    parts = sc_call(boxes_flat, gt_exp)

    sums = jnp.sum(parts, axis=-1)
    per_img = jnp.sum(jnp.reshape(sums, (_B, 2, 3)), axis=1)
    det, cnt, scf = per_img[:, 0], per_img[:, 1], per_img[:, 2]
    any_v = cnt > 0.0
    det_i = jnp.where(any_v, det, 0.0)
    max_probs = jnp.where(any_v, scf / jnp.maximum(cnt, 1.0), 0.0)
    return jnp.mean(det_i), max_probs


# speedup vs baseline: 3.5376x; 3.5376x over previous
import functools

import jax
import jax.numpy as jnp
from jax import lax
from jax.experimental import pallas as pl
from jax.experimental.pallas import tpu as pltpu
from jax.experimental.pallas import tpu_sc as plsc

_FIGSIZE = 416.0
_CONF_THRESH = 0.2
_B = 16
_N = 20000
_HALF = _N // 2
_CB = 2000
_NCH = _HALF // _CB
_GROUPS = _CB // 16
_FIELDS = (0, 1, 2, 3, 4, 6)

_LN2 = 0.6931471805599453
_SQRT2 = 1.4142135623730951


def _log_poly(t):
    bits = lax.bitcast_convert_type(t, jnp.int32)
    e = (bits >> 23) - 127
    m = lax.bitcast_convert_type(
        (bits & 0x007FFFFF) | 0x3F800000, jnp.float32
    )
    big = m > _SQRT2
    m = jnp.where(big, m * 0.5, m)
    ef = e.astype(jnp.float32) + jnp.where(big, 1.0, 0.0)
    z = (m - 1.0) / (m + 1.0)
    z2 = z * z
    q = (0.14285714 * z2 + 0.2) * z2 + 0.33333333
    log_m = (2.0 * z) * (1.0 + z2 * q)
    return ef * _LN2 + log_m


def _worker_body(boxes_hbm, gt_hbm, out_hbm, buf0, buf1, gtv, stg, sem0, sem1):
    img = lax.axis_index("s")
    half = lax.axis_index("c")
    wid = img * 2 + half

    pltpu.sync_copy(gt_hbm.at[img], gtv)
    gx1 = gtv[0]
    gy1 = gtv[1]
    gx2 = gtv[2]
    gy2 = gtv[3]
    area_g = gtv[4]
    thr = gtv[5]

    bufs = (buf0, buf1)
    sems = (sem0, sem1)
    copies = [None, None]

    def start(c):
        n_c = half * _HALF + c * _CB
        cps = []
        for r, f in enumerate(_FIELDS):
            off = pl.multiple_of(img * (7 * _N) + f * _N + n_c, 8)
            cps.append(
                pltpu.async_copy(
                    boxes_hbm.at[pl.ds(off, _CB)],
                    bufs[c % 2].at[pl.ds(r * _CB, _CB)],
                    sems[c % 2],
                )
            )
        copies[c % 2] = cps

    def chunk_body(buf, accs):
        def group(g, accs):
            det, cnt, scf = accs
            o = g * 16
            x = buf[pl.ds(o, 16)]
            y = buf[pl.ds(_CB + o, 16)]
            w = buf[pl.ds(2 * _CB + o, 16)]
            h = buf[pl.ds(3 * _CB + o, 16)]
            cf = buf[pl.ds(4 * _CB + o, 16)]
            cl = buf[pl.ds(5 * _CB + o, 16)]

            wh = w * 0.5
            hh = h * 0.5
            bx1 = x - wh
            by1 = y - hh
            bx2 = x + wh
            by2 = y + hh
            iw = jnp.maximum(jnp.minimum(bx2, gx2) - jnp.maximum(bx1, gx1), 0.0)
            ih = jnp.maximum(jnp.minimum(by2, gy2) - jnp.maximum(by1, gy1), 0.0)
            inter = iw * ih
            union = w * h + area_g - inter
            iou = inter / union
            valid = (iou >= thr) & (cl == 0.0) & (cf > _CONF_THRESH)
            s = jnp.minimum(cf, 1.0 - 1e-6)
            lg = _log_poly(1.0 - s)
            zero = jnp.zeros((16,), jnp.float32)
            det = det - jnp.where(valid, lg * iou, zero)
            cnt = cnt + jnp.where(valid, 1.0, 0.0)
            scf = scf + jnp.where(valid, cf, zero)
            return det, cnt, scf

        return lax.fori_loop(0, _GROUPS, group, accs)

    accs = (
        jnp.zeros((16,), jnp.float32),
        jnp.zeros((16,), jnp.float32),
        jnp.zeros((16,), jnp.float32),
    )
    start(0)
    for c in range(_NCH):
        if c + 1 < _NCH:
            start(c + 1)
        for cp in copies[c % 2]:
            cp.wait()
        accs = chunk_body(bufs[c % 2], accs)

    stg[0, :] = accs[0]
    stg[1, :] = accs[1]
    stg[2, :] = accs[2]
    pltpu.sync_copy(stg, out_hbm.at[wid])


def kernel(boxes, gt, iou_thresh):
    gtn = gt * (1.0 / _FIGSIZE)
    area_g = (gtn[:, 2] - gtn[:, 0]) * (gtn[:, 3] - gtn[:, 1])
    thr = jnp.broadcast_to(jnp.asarray(iou_thresh, jnp.float32), (_B,))
    gt_rows = jnp.concatenate(
        [gtn, area_g[:, None], thr[:, None]], axis=1
    )
    gt_exp = jnp.broadcast_to(gt_rows[:, :, None], (_B, 6, 16)).astype(jnp.float32)

    boxes_flat = jnp.reshape(jnp.transpose(boxes, (0, 2, 1)), (_B * _N * 7,))

    mesh = plsc.VectorSubcoreMesh(core_axis_name="c", subcore_axis_name="s")
    sc_call = functools.partial(
        pl.kernel,
        mesh=mesh,
        out_type=jax.ShapeDtypeStruct((32, 3, 16), jnp.float32),
        scratch_types=[
            pltpu.VMEM((6 * _CB,), jnp.float32),
            pltpu.VMEM((6 * _CB,), jnp.float32),
            pltpu.VMEM((6, 16), jnp.float32),
            pltpu.VMEM((3, 16), jnp.float32),
            pltpu.SemaphoreType.DMA,
            pltpu.SemaphoreType.DMA,
        ],
        compiler_params=pltpu.CompilerParams(needs_layout_passes=False),
    )(_worker_body)
    parts = sc_call(boxes_flat, gt_exp)

    sums = jnp.sum(parts, axis=-1)
    per_img = jnp.sum(jnp.reshape(sums, (_B, 2, 3)), axis=1)
    det, cnt, scf = per_img[:, 0], per_img[:, 1], per_img[:, 2]
    any_v = cnt > 0.0
    det_i = jnp.where(any_v, det, 0.0)
    max_probs = jnp.where(any_v, scf / jnp.maximum(cnt, 1.0), 0.0)
    return jnp.mean(det_i), max_probs


# speedup vs baseline: 5.0775x; 1.4353x over previous
import jax
import jax.numpy as jnp
from jax.experimental import pallas as pl
from jax.experimental.pallas import tpu as pltpu

_FIGSIZE = 416.0
_CONF_THRESH = 0.2
_B = 16
_N = 20000
_CHUNK = 2048
_GRID = (_N + _CHUNK - 1) // _CHUNK


def _body(thr_ref, bt_ref, gt_ref, loss_ref, probs_ref, sdet, scnt, sconf):
    i = pl.program_id(0)

    @pl.when(i == 0)
    def _init():
        sdet[...] = jnp.zeros_like(sdet)
        scnt[...] = jnp.zeros_like(scnt)
        sconf[...] = jnp.zeros_like(sconf)

    x = bt_ref[:, 0, :]
    y = bt_ref[:, 1, :]
    w = bt_ref[:, 2, :]
    h = bt_ref[:, 3, :]
    conf = bt_ref[:, 4, :]
    cls_id = bt_ref[:, 6, :]

    wh = w * 0.5
    hh = h * 0.5
    bx1 = (x - wh) * _FIGSIZE
    by1 = (y - hh) * _FIGSIZE
    bx2 = (x + wh) * _FIGSIZE
    by2 = (y + hh) * _FIGSIZE

    gx1 = gt_ref[:, 0:1]
    gy1 = gt_ref[:, 1:2]
    gx2 = gt_ref[:, 2:3]
    gy2 = gt_ref[:, 3:4]

    ix1 = jnp.maximum(bx1, gx1)
    iy1 = jnp.maximum(by1, gy1)
    ix2 = jnp.minimum(bx2, gx2)
    iy2 = jnp.minimum(by2, gy2)
    inter = jnp.clip(ix2 - ix1, 0.0) * jnp.clip(iy2 - iy1, 0.0)
    area_b = jnp.clip(bx2 - bx1, 0.0) * jnp.clip(by2 - by1, 0.0)
    area_g = (gx2 - gx1) * (gy2 - gy1)
    ious = inter / (area_b + area_g - inter + 1e-9)

    thr = thr_ref[0]
    lane = jax.lax.broadcasted_iota(jnp.int32, (_B, _CHUNK), 1)
    in_bounds = (i * _CHUNK + lane) < _N
    valid = (ious >= thr) & (cls_id == 0.0) & (conf > _CONF_THRESH) & in_bounds

    s = jnp.clip(conf, 1e-6, 1.0 - 1e-6)
    term = -jnp.log1p(-s) * ious

    zero = jnp.zeros_like(term)
    sdet[...] += jnp.sum(jnp.where(valid, term, zero), axis=1, keepdims=True)
    scnt[...] += jnp.sum(jnp.where(valid, 1.0, 0.0), axis=1, keepdims=True)
    sconf[...] += jnp.sum(jnp.where(valid, conf, zero), axis=1, keepdims=True)

    @pl.when(i == _GRID - 1)
    def _fin():
        det = sdet[...]
        cnt = scnt[...]
        sc = sconf[...]
        any_v = cnt > 0.0
        det_i = jnp.where(any_v, det, 0.0)
        probs_ref[...] = jnp.where(any_v, sc / jnp.maximum(cnt, 1.0), 0.0)
        loss_ref[...] = jnp.sum(det_i, keepdims=True) * (1.0 / _B)


def kernel(boxes, gt, iou_thresh):
    bt = jnp.transpose(boxes, (0, 2, 1))
    thr = jnp.reshape(jnp.asarray(iou_thresh, jnp.float32), (1,))

    loss, probs = pl.pallas_call(
        _body,
        grid=(_GRID,),
        in_specs=[
            pl.BlockSpec(memory_space=pltpu.SMEM),
            pl.BlockSpec((_B, 7, _CHUNK), lambda i: (0, 0, i)),
            pl.BlockSpec((_B, 4), lambda i: (0, 0)),
        ],
        out_specs=[
            pl.BlockSpec((1, 1), lambda i: (0, 0)),
            pl.BlockSpec((_B, 1), lambda i: (0, 0)),
        ],
        out_shape=[
            jax.ShapeDtypeStruct((1, 1), jnp.float32),
            jax.ShapeDtypeStruct((_B, 1), jnp.float32),
        ],
        scratch_shapes=[
            pltpu.VMEM((_B, 1), jnp.float32),
            pltpu.VMEM((_B, 1), jnp.float32),
            pltpu.VMEM((_B, 1), jnp.float32),
        ],
    )(thr, bt, gt)
    return jnp.reshape(loss, ()), jnp.reshape(probs, (_B,))


# speedup vs baseline: 15.0612x; 2.9663x over previous
import jax
import jax.numpy as jnp
from jax.experimental import pallas as pl
from jax.experimental.pallas import tpu as pltpu

_FIGSIZE = 416.0
_CONF_THRESH = 0.2
_B = 16
_N = 20000
_CHUNK = 2048
_GRID = (_N + _CHUNK - 1) // _CHUNK


def _body(thr_ref, bt_ref, gt_ref, loss_ref, probs_ref, sdet, scnt, sconf):
    i = pl.program_id(0)

    @pl.when(i == 0)
    def _init():
        sdet[...] = jnp.zeros_like(sdet)
        scnt[...] = jnp.zeros_like(scnt)
        sconf[...] = jnp.zeros_like(sconf)

    x = bt_ref[0]
    y = bt_ref[1]
    w = bt_ref[2]
    h = bt_ref[3]
    conf = bt_ref[4]
    cls_id = bt_ref[6]

    wh = w * 0.5
    hh = h * 0.5
    bx1 = (x - wh) * _FIGSIZE
    by1 = (y - hh) * _FIGSIZE
    bx2 = (x + wh) * _FIGSIZE
    by2 = (y + hh) * _FIGSIZE

    gx1 = gt_ref[:, 0:1]
    gy1 = gt_ref[:, 1:2]
    gx2 = gt_ref[:, 2:3]
    gy2 = gt_ref[:, 3:4]

    ix1 = jnp.maximum(bx1, gx1)
    iy1 = jnp.maximum(by1, gy1)
    ix2 = jnp.minimum(bx2, gx2)
    iy2 = jnp.minimum(by2, gy2)
    inter = jnp.clip(ix2 - ix1, 0.0) * jnp.clip(iy2 - iy1, 0.0)
    area_b = jnp.clip(bx2 - bx1, 0.0) * jnp.clip(by2 - by1, 0.0)
    area_g = (gx2 - gx1) * (gy2 - gy1)
    ious = inter / (area_b + area_g - inter + 1e-9)

    thr = thr_ref[0]
    lane = jax.lax.broadcasted_iota(jnp.int32, (_B, _CHUNK), 1)
    in_bounds = (i * _CHUNK + lane) < _N
    valid = (ious >= thr) & (cls_id == 0.0) & (conf > _CONF_THRESH) & in_bounds

    s = jnp.clip(conf, 1e-6, 1.0 - 1e-6)
    term = -jnp.log1p(-s) * ious

    zero = jnp.zeros_like(term)
    sdet[...] += jnp.sum(jnp.where(valid, term, zero), axis=1, keepdims=True)
    scnt[...] += jnp.sum(jnp.where(valid, 1.0, 0.0), axis=1, keepdims=True)
    sconf[...] += jnp.sum(jnp.where(valid, conf, zero), axis=1, keepdims=True)

    @pl.when(i == _GRID - 1)
    def _fin():
        det = sdet[...]
        cnt = scnt[...]
        sc = sconf[...]
        any_v = cnt > 0.0
        det_i = jnp.where(any_v, det, 0.0)
        probs_ref[...] = jnp.where(any_v, sc / jnp.maximum(cnt, 1.0), 0.0)
        loss_ref[...] = jnp.sum(det_i, keepdims=True) * (1.0 / _B)


def kernel(boxes, gt, iou_thresh):
    bt = jnp.transpose(boxes, (2, 0, 1))
    thr = jnp.reshape(jnp.asarray(iou_thresh, jnp.float32), (1,))

    loss, probs = pl.pallas_call(
        _body,
        grid=(_GRID,),
        in_specs=[
            pl.BlockSpec(memory_space=pltpu.SMEM),
            pl.BlockSpec((7, _B, _CHUNK), lambda i: (0, 0, i)),
            pl.BlockSpec((_B, 4), lambda i: (0, 0)),
        ],
        out_specs=[
            pl.BlockSpec((1, 1), lambda i: (0, 0)),
            pl.BlockSpec((_B, 1), lambda i: (0, 0)),
        ],
        out_shape=[
            jax.ShapeDtypeStruct((1, 1), jnp.float32),
            jax.ShapeDtypeStruct((_B, 1), jnp.float32),
        ],
        scratch_shapes=[
            pltpu.VMEM((_B, 1), jnp.float32),
            pltpu.VMEM((_B, 1), jnp.float32),
            pltpu.VMEM((_B, 1), jnp.float32),
        ],
    )(thr, bt, gt)
    return jnp.reshape(loss, ()), jnp.reshape(probs, (_B,))


# speedup vs baseline: 19.1781x; 1.2733x over previous
import jax
import jax.numpy as jnp
from jax.experimental import pallas as pl
from jax.experimental.pallas import tpu as pltpu

_FIGSIZE = 416.0
_CONF_THRESH = 0.2
_B = 16
_N = 20000
_CHUNK = 5120
_GRID = (_N + _CHUNK - 1) // _CHUNK
_EPS = 1e-9 / (_FIGSIZE * _FIGSIZE)


def _body(thr_ref, bt_ref, cls_ref, gt_ref, loss_ref, probs_ref, sdet, scnt, sconf):
    i = pl.program_id(0)

    @pl.when(i == 0)
    def _init():
        sdet[...] = jnp.zeros_like(sdet)
        scnt[...] = jnp.zeros_like(scnt)
        sconf[...] = jnp.zeros_like(sconf)

    x = bt_ref[0]
    y = bt_ref[1]
    w = bt_ref[2]
    h = bt_ref[3]
    conf = bt_ref[4]
    cls_id = cls_ref[0]

    wh = w * 0.5
    hh = h * 0.5
    bx1 = x - wh
    by1 = y - hh
    bx2 = x + wh
    by2 = y + hh

    gx1 = gt_ref[:, 0:1]
    gy1 = gt_ref[:, 1:2]
    gx2 = gt_ref[:, 2:3]
    gy2 = gt_ref[:, 3:4]
    area_g = gt_ref[:, 4:5]

    iw = jnp.minimum(bx2, gx2) - jnp.maximum(bx1, gx1)
    ih = jnp.minimum(by2, gy2) - jnp.maximum(by1, gy1)
    inter = jnp.clip(iw, 0.0) * jnp.clip(ih, 0.0)
    area_b = jnp.clip(bx2 - bx1, 0.0) * jnp.clip(by2 - by1, 0.0)
    ious = inter / (area_b + area_g - inter + _EPS)

    thr = thr_ref[0]
    lane = jax.lax.broadcasted_iota(jnp.int32, (_B, _CHUNK), 1)
    in_bounds = (i * _CHUNK + lane) < _N
    valid = (ious >= thr) & (cls_id == 0.0) & (conf > _CONF_THRESH) & in_bounds

    s = jnp.minimum(conf, 1.0 - 1e-6)
    term = -jnp.log1p(-s) * ious

    zero = jnp.zeros_like(term)
    sdet[...] += jnp.sum(jnp.where(valid, term, zero), axis=1, keepdims=True)
    scnt[...] += jnp.sum(jnp.where(valid, 1.0, 0.0), axis=1, keepdims=True)
    sconf[...] += jnp.sum(jnp.where(valid, conf, zero), axis=1, keepdims=True)

    @pl.when(i == _GRID - 1)
    def _fin():
        det = sdet[...]
        cnt = scnt[...]
        sc = sconf[...]
        any_v = cnt > 0.0
        det_i = jnp.where(any_v, det, 0.0)
        probs_ref[...] = jnp.where(any_v, sc / jnp.maximum(cnt, 1.0), 0.0)
        loss_ref[...] = jnp.sum(det_i, keepdims=True) * (1.0 / _B)


def kernel(boxes, gt, iou_thresh):
    bt = jnp.transpose(boxes, (2, 0, 1))
    thr = jnp.reshape(jnp.asarray(iou_thresh, jnp.float32), (1,))
    gtn = gt * (1.0 / _FIGSIZE)
    area_g = (gtn[:, 2] - gtn[:, 0]) * (gtn[:, 3] - gtn[:, 1])
    gtx = jnp.concatenate([gtn, area_g[:, None]], axis=1)

    loss, probs = pl.pallas_call(
        _body,
        grid=(_GRID,),
        in_specs=[
            pl.BlockSpec(memory_space=pltpu.SMEM),
            pl.BlockSpec((5, _B, _CHUNK), lambda i: (0, 0, i)),
            pl.BlockSpec((1, _B, _CHUNK), lambda i: (6, 0, i)),
            pl.BlockSpec((_B, 5), lambda i: (0, 0)),
        ],
        out_specs=[
            pl.BlockSpec((1, 1), lambda i: (0, 0)),
            pl.BlockSpec((_B, 1), lambda i: (0, 0)),
        ],
        out_shape=[
            jax.ShapeDtypeStruct((1, 1), jnp.float32),
            jax.ShapeDtypeStruct((_B, 1), jnp.float32),
        ],
        scratch_shapes=[
            pltpu.VMEM((_B, 1), jnp.float32),
            pltpu.VMEM((_B, 1), jnp.float32),
            pltpu.VMEM((_B, 1), jnp.float32),
        ],
    )(thr, bt, bt, gtx)
    return jnp.reshape(loss, ()), jnp.reshape(probs, (_B,))
